# Initial kernel scaffold; baseline (speedup 1.0000x reference)
import functools

import jax
import jax.numpy as jnp
from jax import lax
from jax.experimental import pallas as pl
from jax.experimental.pallas import tpu as pltpu
from jax.experimental.pallas import tpu_sc as plsc

NC = 2
NS = 16
NW = NC * NS
K = 128


def _make_count_kernel(NP, C):
    stripe = NP // NS
    mesh = plsc.VectorSubcoreMesh(core_axis_name="c", subcore_axis_name="s")

    @functools.partial(
        pl.kernel,
        out_type=jax.ShapeDtypeStruct((NC, NP), jnp.float32),
        mesh=mesh,
        scratch_types=[
            pltpu.VMEM((C, K), jnp.int32),
            pltpu.VMEM((stripe,), jnp.float32),
            pltpu.VMEM((K,), jnp.float32),
            pltpu.VMEM_SHARED((NP,), jnp.float32),
        ],
    )
    def count_kernel(cidx_hbm, cnt_hbm, cidx_v, zb_v, ones_v, acc):
        cid = lax.axis_index("c")
        sid = lax.axis_index("s")
        wid = cid * NS + sid

        def zfill(i, carry):
            zb_v[pl.ds(i * 16, 16)] = jnp.zeros((16,), jnp.float32)
            return carry

        lax.fori_loop(0, stripe // 16, zfill, 0)
        for i in range(K // 16):
            ones_v[pl.ds(i * 16, 16)] = jnp.ones((16,), jnp.float32)
        pltpu.sync_copy(zb_v, acc.at[pl.ds(sid * stripe, stripe)])
        plsc.subcore_barrier()

        pltpu.sync_copy(cidx_hbm.at[wid], cidx_v)

        def body(j, carry):
            pltpu.sync_copy(ones_v, acc.at[cidx_v.at[j]], add=True)
            return carry

        lax.fori_loop(0, C, body, 0)
        plsc.subcore_barrier()
        pltpu.sync_copy(acc.at[pl.ds(sid * stripe, stripe)],
                        cnt_hbm.at[cid, pl.ds(sid * stripe, stripe)])

    return count_kernel


def _make_scatter_kernel(D, NP, C):
    stripe = NP // NS
    mesh = plsc.VectorSubcoreMesh(core_axis_name="c", subcore_axis_name="s")

    @functools.partial(
        pl.kernel,
        out_type=jax.ShapeDtypeStruct((NC, NP, D), jnp.float32),
        mesh=mesh,
        scratch_types=[
            pltpu.VMEM((C, K), jnp.int32),
            pltpu.VMEM((C, K), jnp.int32),
            pltpu.VMEM((K, D), jnp.float32),
            pltpu.VMEM((K, D), jnp.float32),
            pltpu.VMEM_SHARED((NP, D), jnp.float32),
            pltpu.SemaphoreType.DMA,
            pltpu.SemaphoreType.DMA,
        ],
    )
    def scatter_kernel(g_hbm, ridx_hbm, cidx_hbm, out_hbm,
                       ridx_v, cidx_v, gb0, gb1, acc, sem0, sem1):
        cid = lax.axis_index("c")
        sid = lax.axis_index("s")
        wid = cid * NS + sid

        def zrow(r, carry):
            for cc in range(D // 16):
                gb0[r, pl.ds(cc * 16, 16)] = jnp.zeros((16,), jnp.float32)
            return carry

        lax.fori_loop(0, K, zrow, 0)
        for t in range(stripe // K):
            pltpu.sync_copy(gb0, acc.at[pl.ds(sid * stripe + t * K, K)])
        plsc.subcore_barrier()

        pltpu.sync_copy(ridx_hbm.at[wid], ridx_v)
        pltpu.sync_copy(cidx_hbm.at[wid], cidx_v)

        def body(jj, carry):
            j0 = jj * 2
            cp0 = pltpu.async_copy(g_hbm.at[ridx_v.at[j0]], gb0, sem0)
            cp1 = pltpu.async_copy(g_hbm.at[ridx_v.at[j0 + 1]], gb1, sem1)
            cp0.wait()
            pltpu.sync_copy(gb0, acc.at[cidx_v.at[j0]], add=True)
            cp1.wait()
            pltpu.sync_copy(gb1, acc.at[cidx_v.at[j0 + 1]], add=True)
            return carry

        lax.fori_loop(0, C // 2, body, 0)
        plsc.subcore_barrier()
        for t in range(stripe // K):
            pltpu.sync_copy(acc.at[pl.ds(sid * stripe + t * K, K)],
                            out_hbm.at[cid, pl.ds(sid * stripe + t * K, K)])

    return scatter_kernel


def _tc_transform(x, W, cnt3):
    N, D = x.shape
    BN = 1000
    grid = N // BN

    def body(x_ref, w_ref, cnt_ref, g_ref, dinv_ref):
        deg = cnt_ref[0] + cnt_ref[1] + 2.0
        dinv = lax.rsqrt(deg)
        h = jnp.dot(x_ref[...], w_ref[...], preferred_element_type=jnp.float32)
        g_ref[...] = h * dinv
        dinv_ref[...] = dinv

    return pl.pallas_call(
        body,
        grid=(grid,),
        in_specs=[
            pl.BlockSpec((BN, D), lambda i: (i, 0)),
            pl.BlockSpec((D, D), lambda i: (0, 0)),
            pl.BlockSpec((NC, BN, 1), lambda i: (0, i, 0)),
        ],
        out_specs=[
            pl.BlockSpec((BN, D), lambda i: (i, 0)),
            pl.BlockSpec((BN, 1), lambda i: (i, 0)),
        ],
        out_shape=[
            jax.ShapeDtypeStruct((N, D), jnp.float32),
            jax.ShapeDtypeStruct((N, 1), jnp.float32),
        ],
    )(x, W, cnt3)


def _tc_finish(s, g, x, dinv, b2, gamma2, beta2):
    N, D = x.shape

    def body(s_ref, g_ref, x_ref, dinv_ref, b_ref, gm_ref, bt_ref, out_ref):
        ssum = s_ref[0, :N, :] + s_ref[1, :N, :]
        agg = dinv_ref[...] * (ssum + 2.0 * g_ref[...]) + b_ref[...]
        mean = jnp.mean(agg, axis=0, keepdims=True)
        cent = agg - mean
        var = jnp.mean(cent * cent, axis=0, keepdims=True)
        y = cent * lax.rsqrt(var + 1e-5) * gm_ref[...] + bt_ref[...]
        out_ref[...] = jnp.maximum(y, 0.0) + x_ref[...]

    return pl.pallas_call(
        body,
        out_shape=jax.ShapeDtypeStruct((N, D), jnp.float32),
    )(s, g, x, dinv, b2, gamma2, beta2)


def kernel(x, edge_index, W, b, gamma, beta):
    N, D = x.shape
    E = edge_index.shape[1]
    NP = 10240
    EPW = -(-E // NW)
    C = -(-EPW // K)
    C += C % 2
    E_PAD = NW * C * K

    pad = E_PAD - E
    row = jnp.concatenate([edge_index[0],
                           jnp.zeros((pad,), edge_index.dtype)]).reshape(NW, C, K)
    col = jnp.concatenate([edge_index[1],
                           jnp.full((pad,), N, edge_index.dtype)]).reshape(NW, C, K)

    cnt = _make_count_kernel(NP, C)(col)
    g, dinv = _tc_transform(x, W, cnt.reshape(NC, NP, 1)[:, :N, :])
    s = _make_scatter_kernel(D, NP, C)(g, row, col)
    return _tc_finish(s, g, x, dinv,
                      b.reshape(1, D), gamma.reshape(1, D), beta.reshape(1, D))


# speedup vs baseline: 10.8830x; 10.8830x over previous
import functools

import jax
import jax.numpy as jnp
from jax import lax
from jax.experimental import pallas as pl
from jax.experimental.pallas import tpu as pltpu
from jax.experimental.pallas import tpu_sc as plsc

NC = 2
NS = 16
NW = NC * NS
K = 128


def _make_count_kernel(NP, C):
    stripe = NP // NS
    mesh = plsc.VectorSubcoreMesh(core_axis_name="c", subcore_axis_name="s")

    @functools.partial(
        pl.kernel,
        out_type=jax.ShapeDtypeStruct((NC, NP), jnp.float32),
        mesh=mesh,
        scratch_types=[
            pltpu.VMEM((C, K), jnp.int32),
            pltpu.VMEM((stripe,), jnp.float32),
            pltpu.VMEM((K,), jnp.float32),
            pltpu.VMEM_SHARED((NP,), jnp.float32),
        ],
    )
    def count_kernel(cidx_hbm, cnt_hbm, cidx_v, zb_v, ones_v, acc):
        cid = lax.axis_index("c")
        sid = lax.axis_index("s")
        wid = cid * NS + sid

        def zfill(i, carry):
            zb_v[pl.ds(i * 16, 16)] = jnp.zeros((16,), jnp.float32)
            return carry

        lax.fori_loop(0, stripe // 16, zfill, 0)
        for i in range(K // 16):
            ones_v[pl.ds(i * 16, 16)] = jnp.ones((16,), jnp.float32)
        pltpu.sync_copy(zb_v, acc.at[pl.ds(sid * stripe, stripe)])
        plsc.subcore_barrier()

        pltpu.sync_copy(cidx_hbm.at[wid], cidx_v)

        def body(j, carry):
            pltpu.sync_copy(ones_v, acc.at[cidx_v.at[j]], add=True)
            return carry

        lax.fori_loop(0, C, body, 0)
        plsc.subcore_barrier()
        pltpu.sync_copy(acc.at[pl.ds(sid * stripe, stripe)],
                        cnt_hbm.at[cid, pl.ds(sid * stripe, stripe)])

    return count_kernel


def _make_scatter_kernel(D, NP, C):
    stripe = NP // NS
    mesh = plsc.VectorSubcoreMesh(core_axis_name="c", subcore_axis_name="s")

    @functools.partial(
        pl.kernel,
        out_type=jax.ShapeDtypeStruct((NC, NP, D), jnp.float32),
        mesh=mesh,
        scratch_types=[
            pltpu.VMEM((C, K), jnp.int32),
            pltpu.VMEM((C, K), jnp.int32),
            pltpu.VMEM((K, D), jnp.float32),
            pltpu.VMEM_SHARED((NP, D), jnp.float32),
            pltpu.SemaphoreType.DMA,
        ],
    )
    def scatter_kernel(g_hbm, ridx_hbm, cidx_hbm, out_hbm,
                       ridx_v, cidx_v, gb0, acc, sem0):
        cid = lax.axis_index("c")
        sid = lax.axis_index("s")
        wid = cid * NS + sid

        def zrow(r, carry):
            for cc in range(D // 16):
                gb0[r, pl.ds(cc * 16, 16)] = jnp.zeros((16,), jnp.float32)
            return carry

        lax.fori_loop(0, K, zrow, 0)
        for t in range(stripe // K):
            pltpu.sync_copy(gb0, acc.at[pl.ds(sid * stripe + t * K, K)])
        plsc.subcore_barrier()

        pltpu.sync_copy(ridx_hbm.at[wid], ridx_v)
        pltpu.sync_copy(cidx_hbm.at[wid], cidx_v)

        def body(j, carry):
            pltpu.async_copy(g_hbm.at[ridx_v.at[j]], gb0, sem0).wait()
            pltpu.sync_copy(gb0, acc.at[cidx_v.at[j]], add=True)
            return carry

        lax.fori_loop(0, C, body, 0)
        plsc.subcore_barrier()
        for t in range(stripe // K):
            pltpu.sync_copy(acc.at[pl.ds(sid * stripe + t * K, K)],
                            out_hbm.at[cid, pl.ds(sid * stripe + t * K, K)])

    return scatter_kernel


def _tc_transform(x, W, cnt3):
    N, D = x.shape
    BN = 1000
    grid = N // BN

    def body(x_ref, w_ref, cnt_ref, g_ref, dinv_ref):
        deg = cnt_ref[0] + cnt_ref[1] + 2.0
        dinv = lax.rsqrt(deg)
        h = jnp.dot(x_ref[...], w_ref[...], preferred_element_type=jnp.float32)
        g_ref[...] = h * dinv
        dinv_ref[...] = dinv

    return pl.pallas_call(
        body,
        grid=(grid,),
        in_specs=[
            pl.BlockSpec((BN, D), lambda i: (i, 0)),
            pl.BlockSpec((D, D), lambda i: (0, 0)),
            pl.BlockSpec((NC, BN, 1), lambda i: (0, i, 0)),
        ],
        out_specs=[
            pl.BlockSpec((BN, D), lambda i: (i, 0)),
            pl.BlockSpec((BN, 1), lambda i: (i, 0)),
        ],
        out_shape=[
            jax.ShapeDtypeStruct((N, D), jnp.float32),
            jax.ShapeDtypeStruct((N, 1), jnp.float32),
        ],
    )(x, W, cnt3)


def _tc_finish(s, g, x, dinv, b2, gamma2, beta2):
    N, D = x.shape

    def body(s_ref, g_ref, x_ref, dinv_ref, b_ref, gm_ref, bt_ref, out_ref):
        ssum = s_ref[0, :N, :] + s_ref[1, :N, :]
        agg = dinv_ref[...] * (ssum + 2.0 * g_ref[...]) + b_ref[...]
        mean = jnp.mean(agg, axis=0, keepdims=True)
        cent = agg - mean
        var = jnp.mean(cent * cent, axis=0, keepdims=True)
        y = cent * lax.rsqrt(var + 1e-5) * gm_ref[...] + bt_ref[...]
        out_ref[...] = jnp.maximum(y, 0.0) + x_ref[...]

    return pl.pallas_call(
        body,
        out_shape=jax.ShapeDtypeStruct((N, D), jnp.float32),
    )(s, g, x, dinv, b2, gamma2, beta2)


def kernel(x, edge_index, W, b, gamma, beta):
    N, D = x.shape
    E = edge_index.shape[1]
    NP = 10240
    EPW = -(-E // NW)
    C = -(-EPW // K)
    C += C % 2
    E_PAD = NW * C * K

    pad = E_PAD - E
    row = jnp.concatenate([edge_index[0],
                           jnp.zeros((pad,), edge_index.dtype)]).reshape(NW, C, K)
    col = jnp.concatenate([edge_index[1],
                           jnp.full((pad,), N, edge_index.dtype)]).reshape(NW, C, K)

    cnt = _make_count_kernel(NP, C)(col)
    g, dinv = _tc_transform(x, W, cnt.reshape(NC, NP, 1)[:, :N, :])
    s = _make_scatter_kernel(D, NP, C)(g, row, col)
    return _tc_finish(s, g, x, dinv,
                      b.reshape(1, D), gamma.reshape(1, D), beta.reshape(1, D))


# speedup vs baseline: 35.5119x; 3.2631x over previous
import functools

import jax
import jax.numpy as jnp
from jax import lax
from jax.experimental import pallas as pl
from jax.experimental.pallas import tpu as pltpu
from jax.experimental.pallas import tpu_sc as plsc

NC = 2
NS = 16
NW = NC * NS
K = 80


def _make_count_kernel(NP, C):
    stripe = NP // NS
    mesh = plsc.VectorSubcoreMesh(core_axis_name="c", subcore_axis_name="s")

    @functools.partial(
        pl.kernel,
        out_type=jax.ShapeDtypeStruct((NC, NP), jnp.float32),
        mesh=mesh,
        scratch_types=[
            pltpu.VMEM((C, K), jnp.int32),
            pltpu.VMEM((stripe,), jnp.float32),
            pltpu.VMEM((K,), jnp.float32),
            pltpu.VMEM_SHARED((NP,), jnp.float32),
        ],
    )
    def count_kernel(cidx_hbm, cnt_hbm, cidx_v, zb_v, ones_v, acc):
        cid = lax.axis_index("c")
        sid = lax.axis_index("s")
        wid = cid * NS + sid

        def zfill(i, carry):
            zb_v[pl.ds(i * 16, 16)] = jnp.zeros((16,), jnp.float32)
            return carry

        lax.fori_loop(0, stripe // 16, zfill, 0)
        for i in range(K // 16):
            ones_v[pl.ds(i * 16, 16)] = jnp.ones((16,), jnp.float32)
        pltpu.sync_copy(zb_v, acc.at[pl.ds(sid * stripe, stripe)])
        plsc.subcore_barrier()

        pltpu.sync_copy(cidx_hbm.at[wid], cidx_v)

        def body(j, carry):
            pltpu.sync_copy(ones_v, acc.at[cidx_v.at[j]], add=True)
            return carry

        lax.fori_loop(0, C, body, 0)
        plsc.subcore_barrier()
        pltpu.sync_copy(acc.at[pl.ds(sid * stripe, stripe)],
                        cnt_hbm.at[cid, pl.ds(sid * stripe, stripe)])

    return count_kernel


def _make_scatter_kernel(D, NP, C):
    stripe = NP // NS
    mesh = plsc.VectorSubcoreMesh(core_axis_name="c", subcore_axis_name="s")

    @functools.partial(
        pl.kernel,
        out_type=jax.ShapeDtypeStruct((NC, NP, D), jnp.float32),
        mesh=mesh,
        scratch_types=[
            pltpu.VMEM((C * K,), jnp.int32),
            pltpu.VMEM((C, K), jnp.int32),
            pltpu.VMEM((K, D), jnp.float32),
            pltpu.VMEM((K, D), jnp.float32),
            pltpu.VMEM_SHARED((NP, D), jnp.float32),
            pltpu.SemaphoreType.DMA,
            pltpu.SemaphoreType.DMA,
        ],
    )
    def scatter_kernel(g_hbm, ridx_hbm, cidx_hbm, out_hbm,
                       ridx_v, cidx_v, gb0, gb1, acc, sem0, sem1):
        cid = lax.axis_index("c")
        sid = lax.axis_index("s")
        wid = cid * NS + sid

        def zrow(r, carry):
            for cc in range(D // 16):
                gb0[r, pl.ds(cc * 16, 16)] = jnp.zeros((16,), jnp.float32)
            return carry

        lax.fori_loop(0, K, zrow, 0)
        for t in range(stripe // K):
            pltpu.sync_copy(gb0, acc.at[pl.ds(sid * stripe + t * K, K)])
        plsc.subcore_barrier()

        pltpu.sync_copy(ridx_hbm.at[wid], ridx_v)
        pltpu.sync_copy(cidx_hbm.at[wid], cidx_v)

        def rslice(j):
            return ridx_v.at[pl.ds(j * K, K)]

        pltpu.async_copy(g_hbm.at[rslice(0)], gb0, sem0)

        def body(i, carry):
            j0 = 2 * i
            pltpu.async_copy(g_hbm.at[rslice(j0 + 1)], gb1, sem1)
            pltpu.make_async_copy(g_hbm.at[rslice(j0)], gb0, sem0).wait()
            pltpu.sync_copy(gb0, acc.at[cidx_v.at[j0]], add=True)
            pltpu.async_copy(g_hbm.at[rslice(j0 + 2)], gb0, sem0)
            pltpu.make_async_copy(g_hbm.at[rslice(j0 + 1)], gb1, sem1).wait()
            pltpu.sync_copy(gb1, acc.at[cidx_v.at[j0 + 1]], add=True)
            return carry

        lax.fori_loop(0, (C - 1) // 2, body, 0)
        pltpu.make_async_copy(g_hbm.at[rslice(C - 1)], gb0, sem0).wait()
        pltpu.sync_copy(gb0, acc.at[cidx_v.at[C - 1]], add=True)
        plsc.subcore_barrier()
        for t in range(stripe // K):
            pltpu.sync_copy(acc.at[pl.ds(sid * stripe + t * K, K)],
                            out_hbm.at[cid, pl.ds(sid * stripe + t * K, K)])

    return scatter_kernel


def _tc_transform(x, W, cnt3):
    N, D = x.shape
    BN = 1000
    grid = N // BN

    def body(x_ref, w_ref, cnt_ref, g_ref, dinv_ref):
        deg = cnt_ref[0] + cnt_ref[1] + 2.0
        dinv = lax.rsqrt(deg)
        h = jnp.dot(x_ref[...], w_ref[...], preferred_element_type=jnp.float32)
        g_ref[...] = h * dinv
        dinv_ref[...] = dinv

    return pl.pallas_call(
        body,
        grid=(grid,),
        in_specs=[
            pl.BlockSpec((BN, D), lambda i: (i, 0)),
            pl.BlockSpec((D, D), lambda i: (0, 0)),
            pl.BlockSpec((NC, BN, 1), lambda i: (0, i, 0)),
        ],
        out_specs=[
            pl.BlockSpec((BN, D), lambda i: (i, 0)),
            pl.BlockSpec((BN, 1), lambda i: (i, 0)),
        ],
        out_shape=[
            jax.ShapeDtypeStruct((N, D), jnp.float32),
            jax.ShapeDtypeStruct((N, 1), jnp.float32),
        ],
    )(x, W, cnt3)


def _tc_finish(s, g, x, dinv, b2, gamma2, beta2):
    N, D = x.shape

    def body(s_ref, g_ref, x_ref, dinv_ref, b_ref, gm_ref, bt_ref, out_ref):
        ssum = s_ref[0, :N, :] + s_ref[1, :N, :]
        agg = dinv_ref[...] * (ssum + 2.0 * g_ref[...]) + b_ref[...]
        mean = jnp.mean(agg, axis=0, keepdims=True)
        cent = agg - mean
        var = jnp.mean(cent * cent, axis=0, keepdims=True)
        y = cent * lax.rsqrt(var + 1e-5) * gm_ref[...] + bt_ref[...]
        out_ref[...] = jnp.maximum(y, 0.0) + x_ref[...]

    return pl.pallas_call(
        body,
        out_shape=jax.ShapeDtypeStruct((N, D), jnp.float32),
    )(s, g, x, dinv, b2, gamma2, beta2)


def kernel(x, edge_index, W, b, gamma, beta):
    N, D = x.shape
    E = edge_index.shape[1]
    NP = 10240
    EPW = E // NW
    C = EPW // K

    row = edge_index[0].reshape(NW, C * K)
    col = edge_index[1].reshape(NW, C, K)

    cnt = _make_count_kernel(NP, C)(col)
    g, dinv = _tc_transform(x, W, cnt.reshape(NC, NP, 1)[:, :N, :])
    s = _make_scatter_kernel(D, NP, C)(g, row, col)
    return _tc_finish(s, g, x, dinv,
                      b.reshape(1, D), gamma.reshape(1, D), beta.reshape(1, D))


# speedup vs baseline: 37.1996x; 1.0475x over previous
import functools

import jax
import jax.numpy as jnp
from jax import lax
from jax.experimental import pallas as pl
from jax.experimental.pallas import tpu as pltpu
from jax.experimental.pallas import tpu_sc as plsc

NC = 2
NS = 16
NW = NC * NS
K = 80


def _make_count_kernel(NP, C):
    stripe = NP // NS
    mesh = plsc.VectorSubcoreMesh(core_axis_name="c", subcore_axis_name="s")

    @functools.partial(
        pl.kernel,
        out_type=jax.ShapeDtypeStruct((NC, NP), jnp.float32),
        mesh=mesh,
        scratch_types=[
            pltpu.VMEM((C, K), jnp.int32),
            pltpu.VMEM((stripe,), jnp.float32),
            pltpu.VMEM((K,), jnp.float32),
            pltpu.VMEM_SHARED((NP,), jnp.float32),
        ],
    )
    def count_kernel(edges_hbm, cnt_hbm, cidx_v, zb_v, ones_v, acc):
        cid = lax.axis_index("c")
        sid = lax.axis_index("s")
        wid = cid * NS + sid

        def zfill(i, carry):
            zb_v[pl.ds(i * 16, 16)] = jnp.zeros((16,), jnp.float32)
            return carry

        lax.fori_loop(0, stripe // 16, zfill, 0)
        for i in range(K // 16):
            ones_v[pl.ds(i * 16, 16)] = jnp.ones((16,), jnp.float32)
        pltpu.sync_copy(zb_v, acc.at[pl.ds(sid * stripe, stripe)])
        plsc.subcore_barrier()

        pltpu.sync_copy(edges_hbm.at[1, wid], cidx_v)

        def body(j, carry):
            pltpu.sync_copy(ones_v, acc.at[cidx_v.at[j]], add=True)
            return carry

        lax.fori_loop(0, C, body, 0)
        plsc.subcore_barrier()
        pltpu.sync_copy(acc.at[pl.ds(sid * stripe, stripe)],
                        cnt_hbm.at[cid, pl.ds(sid * stripe, stripe)])

    return count_kernel


def _make_scatter_kernel(D, NP, C):
    stripe = NP // NS
    mesh = plsc.VectorSubcoreMesh(core_axis_name="c", subcore_axis_name="s")

    @functools.partial(
        pl.kernel,
        out_type=jax.ShapeDtypeStruct((NC, NP, D), jnp.float32),
        mesh=mesh,
        scratch_types=[
            pltpu.VMEM((C * K,), jnp.int32),
            pltpu.VMEM((C, K), jnp.int32),
            pltpu.VMEM((K, D), jnp.float32),
            pltpu.VMEM((K, D), jnp.float32),
            pltpu.VMEM_SHARED((NP, D), jnp.float32),
            pltpu.SemaphoreType.DMA,
            pltpu.SemaphoreType.DMA,
        ],
    )
    def scatter_kernel(g_hbm, erow_hbm, ecol_hbm, out_hbm,
                       ridx_v, cidx_v, gb0, gb1, acc, sem0, sem1):
        cid = lax.axis_index("c")
        sid = lax.axis_index("s")
        wid = cid * NS + sid

        def zrow(r, carry):
            for cc in range(D // 16):
                gb0[r, pl.ds(cc * 16, 16)] = jnp.zeros((16,), jnp.float32)
            return carry

        lax.fori_loop(0, K, zrow, 0)
        for t in range(stripe // K):
            pltpu.sync_copy(gb0, acc.at[pl.ds(sid * stripe + t * K, K)])
        plsc.subcore_barrier()

        pltpu.sync_copy(erow_hbm.at[0, wid], ridx_v)
        pltpu.sync_copy(ecol_hbm.at[1, wid], cidx_v)

        def rslice(j):
            return ridx_v.at[pl.ds(j * K, K)]

        pltpu.async_copy(g_hbm.at[rslice(0)], gb0, sem0)

        def body(i, carry):
            j0 = 2 * i
            pltpu.async_copy(g_hbm.at[rslice(j0 + 1)], gb1, sem1)
            pltpu.make_async_copy(g_hbm.at[rslice(j0)], gb0, sem0).wait()
            pltpu.sync_copy(gb0, acc.at[cidx_v.at[j0]], add=True)
            pltpu.async_copy(g_hbm.at[rslice(j0 + 2)], gb0, sem0)
            pltpu.make_async_copy(g_hbm.at[rslice(j0 + 1)], gb1, sem1).wait()
            pltpu.sync_copy(gb1, acc.at[cidx_v.at[j0 + 1]], add=True)
            return carry

        lax.fori_loop(0, (C - 1) // 2, body, 0)
        pltpu.make_async_copy(g_hbm.at[rslice(C - 1)], gb0, sem0).wait()
        pltpu.sync_copy(gb0, acc.at[cidx_v.at[C - 1]], add=True)
        plsc.subcore_barrier()
        for t in range(stripe // K):
            pltpu.sync_copy(acc.at[pl.ds(sid * stripe + t * K, K)],
                            out_hbm.at[cid, pl.ds(sid * stripe + t * K, K)])

    return scatter_kernel


def _tc_transform(x, W, cnt3):
    N, D = x.shape
    BN = 1000
    grid = N // BN

    def body(x_ref, w_ref, cnt_ref, g_ref, dinv_ref):
        deg = cnt_ref[0] + cnt_ref[1] + 2.0
        dinv = lax.rsqrt(deg)
        h = jnp.dot(x_ref[...], w_ref[...], preferred_element_type=jnp.float32)
        g_ref[...] = h * dinv
        dinv_ref[...] = dinv

    return pl.pallas_call(
        body,
        grid=(grid,),
        in_specs=[
            pl.BlockSpec((BN, D), lambda i: (i, 0)),
            pl.BlockSpec((D, D), lambda i: (0, 0)),
            pl.BlockSpec((NC, BN, 1), lambda i: (0, i, 0)),
        ],
        out_specs=[
            pl.BlockSpec((BN, D), lambda i: (i, 0)),
            pl.BlockSpec((BN, 1), lambda i: (i, 0)),
        ],
        out_shape=[
            jax.ShapeDtypeStruct((N, D), jnp.float32),
            jax.ShapeDtypeStruct((N, 1), jnp.float32),
        ],
    )(x, W, cnt3)


def _tc_finish(s, g, x, dinv, b2, gamma2, beta2):
    N, D = x.shape

    def body(s_ref, g_ref, x_ref, dinv_ref, b_ref, gm_ref, bt_ref, out_ref):
        ssum = s_ref[0, :N, :] + s_ref[1, :N, :]
        agg = dinv_ref[...] * (ssum + 2.0 * g_ref[...]) + b_ref[...]
        mean = jnp.mean(agg, axis=0, keepdims=True)
        cent = agg - mean
        var = jnp.mean(cent * cent, axis=0, keepdims=True)
        y = cent * lax.rsqrt(var + 1e-5) * gm_ref[...] + bt_ref[...]
        out_ref[...] = jnp.maximum(y, 0.0) + x_ref[...]

    return pl.pallas_call(
        body,
        out_shape=jax.ShapeDtypeStruct((N, D), jnp.float32),
    )(s, g, x, dinv, b2, gamma2, beta2)


def kernel(x, edge_index, W, b, gamma, beta):
    N, D = x.shape
    E = edge_index.shape[1]
    NP = 10240
    EPW = E // NW
    C = EPW // K

    e3 = edge_index.reshape(2, NW, C * K)
    e4 = edge_index.reshape(2, NW, C, K)

    cnt = _make_count_kernel(NP, C)(e4)
    g, dinv = _tc_transform(x, W, cnt.reshape(NC, NP, 1))
    s = _make_scatter_kernel(D, NP, C)(g, e3, e4)
    return _tc_finish(s, g, x, dinv,
                      b.reshape(1, D), gamma.reshape(1, D), beta.reshape(1, D))
